# baseline (speedup 1.0000x reference)
import functools

import jax
import jax.numpy as jnp
from jax import lax
from jax.experimental import pallas as pl
from jax.experimental.pallas import tpu as pltpu
from jax.experimental.pallas import tpu_sc as plsc

DIM = 64
NUM_NEG = 5
NLOG = 1 + NUM_NEG
CHUNK = 128
LANES = 16
VB = 128

_SC_PARAMS = pltpu.CompilerParams(
    use_tc_tiling_on_sc=True, needs_layout_passes=False
)


SB = 384


@functools.lru_cache(maxsize=None)
def _build_sc_repack(V: int, NW: int):
    nfull = V // SB
    per_w = (nfull + NW - 1) // NW
    rows_out = V // 2
    tail_rows = (V - nfull * SB) // 2
    mesh = plsc.VectorSubcoreMesh(core_axis_name="c", subcore_axis_name="s")

    @functools.partial(
        pl.kernel,
        mesh=mesh,
        compiler_params=_SC_PARAMS,
        out_type=jax.ShapeDtypeStruct((rows_out, 2 * DIM), jnp.float32),
        scratch_types=[
            pltpu.VMEM((DIM, SB), jnp.float32),
            pltpu.VMEM((DIM, SB), jnp.float32),
            pltpu.VMEM((SB // 2, 2 * DIM), jnp.float32),
            pltpu.VMEM((SB // 2, 2 * DIM), jnp.float32),
            pltpu.VMEM((tail_rows, 2 * DIM), jnp.float32),
            pltpu.SemaphoreType.DMA,
            pltpu.SemaphoreType.DMA,
            pltpu.SemaphoreType.DMA,
            pltpu.SemaphoreType.DMA,
        ],
    )
    def repack(
        tt_hbm, tail_hbm, out_hbm,
        st_a, st_b, ob_a, ob_b, tbuf, sin_a, sin_b, sout_a, sout_b,
    ):
        wid = lax.axis_index("s") * 2 + lax.axis_index("c")
        lane = lax.iota(jnp.int32, LANES)
        base_c = wid * per_w
        my_n = jnp.maximum(0, jnp.minimum(per_w, nfull - base_c))

        def in_cps(t, buf, sem):
            c0 = (base_c + t) * SB
            return [
                pltpu.make_async_copy(
                    tt_hbm.at[pl.ds(h * (DIM // 2), DIM // 2),
                              pl.ds(c0, SB)],
                    buf.at[pl.ds(h * (DIM // 2), DIM // 2), :],
                    sem,
                )
                for h in range(2)
            ]

        def out_cps(t, buf, sem):
            r0 = (base_c + t) * (SB // 2)
            return [
                pltpu.make_async_copy(
                    buf.at[pl.ds(h * (SB // 4), SB // 4), :],
                    out_hbm.at[pl.ds(r0 + h * (SB // 4), SB // 4), :],
                    sem,
                )
                for h in range(2)
            ]

        def start_all(cps):
            for cp in cps:
                cp.start()

        def wait_all(cps):
            for cp in cps:
                cp.wait()

        rows_m = []
        par_m = []
        swz_m = []
        for m in range(SB // LANES):
            c = m * LANES + lane
            rows_m.append(c >> 1)
            par_m.append(c & 1)
            swz_m.append(c & 14)

        def transpose_block(stage, obuf):
            def dstep(d, _):
                for m in range(SB // LANES):
                    v = stage[d, pl.ds(m * LANES, LANES)]
                    col = ((2 * d) ^ swz_m[m]) + par_m[m]
                    plsc.store_scatter(obuf, [rows_m[m], col], v)
                return 0

            lax.fori_loop(0, DIM, dstep, 0, unroll=8)

        @pl.when(0 < my_n)
        def _():
            start_all(in_cps(0, st_a, sin_a))

        def halfstep(t, stage, obuf, sin, sout, nstage, nsin):
            @pl.when(t < my_n)
            def _():
                wait_all(in_cps(t, stage, sin))

                @pl.when(t + 1 < my_n)
                def _():
                    start_all(in_cps(t + 1, nstage, nsin))

                @pl.when(t >= 2)
                def _():
                    wait_all(out_cps(t - 2, obuf, sout))

                transpose_block(stage, obuf)
                start_all(out_cps(t, obuf, sout))

        def step(u, _):
            halfstep(2 * u, st_a, ob_a, sin_a, sout_a, st_b, sin_b)
            halfstep(2 * u + 1, st_b, ob_b, sin_b, sout_b, st_a, sin_a)
            return 0

        lax.fori_loop(0, (per_w + 1) // 2, step, 0)

        def drain(t):
            @pl.when((t >= 0) & (t % 2 == 0))
            def _():
                wait_all(out_cps(t, ob_a, sout_a))

            @pl.when((t >= 0) & (t % 2 == 1))
            def _():
                wait_all(out_cps(t, ob_b, sout_b))

        drain(my_n - 2)
        drain(my_n - 1)

        @pl.when(wid == NW - 1)
        def _():
            pltpu.sync_copy(tail_hbm, tbuf)
            pltpu.sync_copy(
                tbuf, out_hbm.at[pl.ds(nfull * (SB // 2), tail_rows), :]
            )

    return repack


@functools.lru_cache(maxsize=None)
def _build_sc_kernel(B: int, NW: int):
    b_per_w = B // NW
    n_chunks = b_per_w // CHUNK
    mesh = plsc.VectorSubcoreMesh(core_axis_name="c", subcore_axis_name="s")

    @functools.partial(
        pl.kernel,
        mesh=mesh,
        compiler_params=_SC_PARAMS,
        out_type=jax.ShapeDtypeStruct((B * NLOG,), jnp.float32),
        scratch_types=[
            pltpu.VMEM((CHUNK,), jnp.int32),
            pltpu.VMEM((CHUNK,), jnp.int32),
            pltpu.VMEM((CHUNK * NUM_NEG,), jnp.int32),
            pltpu.VMEM((CHUNK,), jnp.int32),
            pltpu.VMEM((CHUNK,), jnp.int32),
            pltpu.VMEM((CHUNK * NUM_NEG,), jnp.int32),
            pltpu.VMEM((CHUNK, 2 * DIM), jnp.float32),
            pltpu.VMEM((CHUNK, 2 * DIM), jnp.float32),
            pltpu.VMEM((CHUNK * NUM_NEG, 2 * DIM), jnp.float32),
            pltpu.VMEM((CHUNK * NLOG,), jnp.float32),
            pltpu.SemaphoreType.DMA,
        ],
    )
    def sc_kernel(
        cen_hbm, pos_hbm, neg_hbm, table_hbm, out_hbm,
        idc, idp, idn, rowc, rowp, rown, rows_c, rows_p, rows_n, out_v, sem,
    ):
        wid = lax.axis_index("s") * 2 + lax.axis_index("c")
        lane = lax.iota(jnp.int32, LANES)

        for c in range(n_chunks):
            base = wid * b_per_w + c * CHUNK
            pltpu.sync_copy(cen_hbm.at[pl.ds(base, CHUNK)], idc)
            pltpu.sync_copy(pos_hbm.at[pl.ds(base, CHUNK)], idp)
            pltpu.sync_copy(
                neg_hbm.at[pl.ds(base * NUM_NEG, CHUNK * NUM_NEG)], idn
            )

            def halve(i, _):
                s = pl.ds(i * LANES, LANES)
                rowc[s] = idc[s] >> 1
                rowp[s] = idp[s] >> 1
                return 0

            lax.fori_loop(0, CHUNK // LANES, halve, 0)

            def halve_n(i, _):
                s = pl.ds(i * LANES, LANES)
                rown[s] = idn[s] >> 1
                return 0

            lax.fori_loop(0, CHUNK * NUM_NEG // LANES, halve_n, 0)

            cps = [
                pltpu.async_copy(table_hbm.at[rowc], rows_c, sem),
                pltpu.async_copy(table_hbm.at[rowp], rows_p, sem),
            ] + [
                pltpu.async_copy(
                    table_hbm.at[rown.at[pl.ds(g * CHUNK, CHUNK)]],
                    rows_n.at[pl.ds(g * CHUNK, CHUNK)],
                    sem,
                )
                for g in range(NUM_NEG)
            ]
            for cp in cps:
                cp.wait()

            def group(g, _):
                bvec = g * LANES + lane
                s = pl.ds(g * LANES, LANES)
                q_c, w_c = idc[s] & 1, idc[s] & 14
                q_p, w_p = idp[s] & 1, idp[s] & 14
                nvecs = [bvec * NUM_NEG + j for j in range(NUM_NEG)]
                idn_j = [
                    plsc.load_gather(idn, [nvecs[j]]) for j in range(NUM_NEG)
                ]
                q_n = [x & 1 for x in idn_j]
                w_n = [x & 14 for x in idn_j]
                acc = [jnp.zeros((LANES,), jnp.float32) for _ in range(NLOG)]
                for d in range(DIM):
                    cen = plsc.load_gather(
                        rows_c, [bvec, (2 * d + q_c) ^ w_c]
                    )
                    acc[0] = acc[0] + cen * plsc.load_gather(
                        rows_p, [bvec, (2 * d + q_p) ^ w_p]
                    )
                    for j in range(NUM_NEG):
                        acc[1 + j] = acc[1 + j] + cen * plsc.load_gather(
                            rows_n, [nvecs[j], (2 * d + q_n[j]) ^ w_n[j]]
                        )
                for j in range(NLOG):
                    prob = 1.0 / (1.0 + jnp.exp(-acc[j]))
                    plsc.store_scatter(out_v, [bvec * NLOG + j], prob)
                return 0

            lax.fori_loop(0, CHUNK // LANES, group, 0)

            pltpu.sync_copy(out_v, out_hbm.at[pl.ds(base * NLOG, CHUNK * NLOG)])

    return sc_kernel


def kernel(x_center, x_positive, x_negative, table):
    B = x_center.shape[0]
    V = table.shape[0]
    NW = 32
    neg_flat = x_negative.reshape(B * NUM_NEG)
    nfull = V // SB
    tl = table[nfull * SB :, :]
    j = jnp.arange(tl.shape[0] // 2)[:, None, None]
    p = jnp.arange(2)[None, :, None]
    d = jnp.arange(DIM)[None, None, :]
    vloc = 2 * j + p
    col = (2 * d + p) ^ (vloc & 14)
    vals = tl[vloc, d]
    tail = (
        jnp.zeros((tl.shape[0] // 2, 2 * DIM), jnp.float32)
        .at[j, col]
        .set(vals)
    )
    table2 = _build_sc_repack(V, NW)(table.T, tail)
    flat = _build_sc_kernel(B, NW)(x_center, x_positive, neg_flat, table2)
    return flat.reshape(B, NLOG)


# speedup vs baseline: 1.0162x; 1.0162x over previous
import functools

import jax
import jax.numpy as jnp
from jax import lax
from jax.experimental import pallas as pl
from jax.experimental.pallas import tpu as pltpu
from jax.experimental.pallas import tpu_sc as plsc

DIM = 64
NUM_NEG = 5
NLOG = 1 + NUM_NEG
CHUNK = 128
LANES = 16
VB = 128

_SC_PARAMS = pltpu.CompilerParams(
    use_tc_tiling_on_sc=True, needs_layout_passes=False
)


SB = 384


@functools.lru_cache(maxsize=None)
def _build_sc_repack(V: int, NW: int):
    nfull = V // SB
    per_w = (nfull + NW - 1) // NW
    rows_out = V // 2
    tail_rows = (V - nfull * SB) // 2
    mesh = plsc.VectorSubcoreMesh(core_axis_name="c", subcore_axis_name="s")

    @functools.partial(
        pl.kernel,
        mesh=mesh,
        compiler_params=_SC_PARAMS,
        out_type=jax.ShapeDtypeStruct((rows_out, 2 * DIM), jnp.float32),
        scratch_types=[
            pltpu.VMEM((DIM, SB), jnp.float32),
            pltpu.VMEM((DIM, SB), jnp.float32),
            pltpu.VMEM((SB // 2, 2 * DIM), jnp.float32),
            pltpu.VMEM((SB // 2, 2 * DIM), jnp.float32),
            pltpu.VMEM((tail_rows, 2 * DIM), jnp.float32),
            pltpu.SemaphoreType.DMA,
            pltpu.SemaphoreType.DMA,
            pltpu.SemaphoreType.DMA,
            pltpu.SemaphoreType.DMA,
        ],
    )
    def repack(
        tt_hbm, tail_hbm, out_hbm,
        st_a, st_b, ob_a, ob_b, tbuf, sin_a, sin_b, sout_a, sout_b,
    ):
        wid = lax.axis_index("s") * 2 + lax.axis_index("c")
        lane = lax.iota(jnp.int32, LANES)
        base_c = wid * per_w
        my_n = jnp.maximum(0, jnp.minimum(per_w, nfull - base_c))

        def in_cps(t, buf, sem):
            c0 = (base_c + t) * SB
            return [
                pltpu.make_async_copy(
                    tt_hbm.at[pl.ds(h * (DIM // 2), DIM // 2),
                              pl.ds(c0, SB)],
                    buf.at[pl.ds(h * (DIM // 2), DIM // 2), :],
                    sem,
                )
                for h in range(2)
            ]

        def out_cps(t, buf, sem):
            r0 = (base_c + t) * (SB // 2)
            return [
                pltpu.make_async_copy(
                    buf.at[pl.ds(h * (SB // 4), SB // 4), :],
                    out_hbm.at[pl.ds(r0 + h * (SB // 4), SB // 4), :],
                    sem,
                )
                for h in range(2)
            ]

        def start_all(cps):
            for cp in cps:
                cp.start()

        def wait_all(cps):
            for cp in cps:
                cp.wait()

        rows_m = []
        par64_m = []
        key_m = []
        for m in range(SB // LANES):
            c = m * LANES + lane
            rows_m.append(c >> 1)
            par64_m.append((c & 1) * DIM)
            key_m.append(c & 15)

        def transpose_block(stage, obuf):
            def dstep(d, _):
                for m in range(SB // LANES):
                    v = stage[d, pl.ds(m * LANES, LANES)]
                    col = par64_m[m] + (d ^ key_m[m])
                    plsc.store_scatter(obuf, [rows_m[m], col], v)
                return 0

            lax.fori_loop(0, DIM, dstep, 0, unroll=8)

        @pl.when(0 < my_n)
        def _():
            start_all(in_cps(0, st_a, sin_a))

        def halfstep(t, stage, obuf, sin, sout, nstage, nsin):
            @pl.when(t < my_n)
            def _():
                wait_all(in_cps(t, stage, sin))

                @pl.when(t + 1 < my_n)
                def _():
                    start_all(in_cps(t + 1, nstage, nsin))

                @pl.when(t >= 2)
                def _():
                    wait_all(out_cps(t - 2, obuf, sout))

                transpose_block(stage, obuf)
                start_all(out_cps(t, obuf, sout))

        def step(u, _):
            halfstep(2 * u, st_a, ob_a, sin_a, sout_a, st_b, sin_b)
            halfstep(2 * u + 1, st_b, ob_b, sin_b, sout_b, st_a, sin_a)
            return 0

        lax.fori_loop(0, (per_w + 1) // 2, step, 0)

        def drain(t):
            @pl.when((t >= 0) & (t % 2 == 0))
            def _():
                wait_all(out_cps(t, ob_a, sout_a))

            @pl.when((t >= 0) & (t % 2 == 1))
            def _():
                wait_all(out_cps(t, ob_b, sout_b))

        drain(my_n - 2)
        drain(my_n - 1)

        @pl.when(wid == NW - 1)
        def _():
            pltpu.sync_copy(tail_hbm, tbuf)
            pltpu.sync_copy(
                tbuf, out_hbm.at[pl.ds(nfull * (SB // 2), tail_rows), :]
            )

    return repack


@functools.lru_cache(maxsize=None)
def _build_sc_kernel(B: int, NW: int):
    b_per_w = B // NW
    n_chunks = b_per_w // CHUNK
    mesh = plsc.VectorSubcoreMesh(core_axis_name="c", subcore_axis_name="s")

    @functools.partial(
        pl.kernel,
        mesh=mesh,
        compiler_params=pltpu.CompilerParams(
            use_tc_tiling_on_sc=False, needs_layout_passes=False
        ),
        out_type=jax.ShapeDtypeStruct((B * NLOG,), jnp.float32),
        scratch_types=[
            pltpu.VMEM((CHUNK,), jnp.int32),
            pltpu.VMEM((CHUNK,), jnp.int32),
            pltpu.VMEM((CHUNK * NUM_NEG,), jnp.int32),
            pltpu.VMEM((CHUNK, DIM), jnp.float32),
            pltpu.VMEM((CHUNK, DIM), jnp.float32),
            pltpu.VMEM((CHUNK * NUM_NEG, DIM), jnp.float32),
            pltpu.VMEM((CHUNK * NLOG,), jnp.float32),
            pltpu.SemaphoreType.DMA,
        ],
    )
    def sc_kernel(
        cen_hbm, pos_hbm, neg_hbm, table_hbm, out_hbm,
        idc, idp, idn, rows_c, rows_p, rows_n, out_v, sem,
    ):
        wid = lax.axis_index("s") * 2 + lax.axis_index("c")
        lane = lax.iota(jnp.int32, LANES)

        for c in range(n_chunks):
            base = wid * b_per_w + c * CHUNK
            pltpu.sync_copy(cen_hbm.at[pl.ds(base, CHUNK)], idc)
            pltpu.sync_copy(pos_hbm.at[pl.ds(base, CHUNK)], idp)
            pltpu.sync_copy(
                neg_hbm.at[pl.ds(base * NUM_NEG, CHUNK * NUM_NEG)], idn
            )

            cps = [
                pltpu.async_copy(table_hbm.at[idc], rows_c, sem),
                pltpu.async_copy(table_hbm.at[idp], rows_p, sem),
            ] + [
                pltpu.async_copy(
                    table_hbm.at[idn.at[pl.ds(g * CHUNK, CHUNK)]],
                    rows_n.at[pl.ds(g * CHUNK, CHUNK)],
                    sem,
                )
                for g in range(NUM_NEG)
            ]
            for cp in cps:
                cp.wait()

            def group(g, _):
                bvec = g * LANES + lane
                s = pl.ds(g * LANES, LANES)
                w_c = idc[s] & 15
                w_p = idp[s] & 15
                nvecs = [bvec * NUM_NEG + j for j in range(NUM_NEG)]
                w_n = [
                    plsc.load_gather(idn, [nvecs[j]]) & 15
                    for j in range(NUM_NEG)
                ]
                acc = [jnp.zeros((LANES,), jnp.float32) for _ in range(NLOG)]
                for d in range(DIM):
                    cen = plsc.load_gather(rows_c, [bvec, d ^ w_c])
                    acc[0] = acc[0] + cen * plsc.load_gather(
                        rows_p, [bvec, d ^ w_p]
                    )
                    for j in range(NUM_NEG):
                        acc[1 + j] = acc[1 + j] + cen * plsc.load_gather(
                            rows_n, [nvecs[j], d ^ w_n[j]]
                        )
                for j in range(NLOG):
                    prob = 1.0 / (1.0 + jnp.exp(-acc[j]))
                    plsc.store_scatter(out_v, [bvec * NLOG + j], prob)
                return 0

            lax.fori_loop(0, CHUNK // LANES, group, 0)

            pltpu.sync_copy(out_v, out_hbm.at[pl.ds(base * NLOG, CHUNK * NLOG)])

    return sc_kernel


def kernel(x_center, x_positive, x_negative, table):
    B = x_center.shape[0]
    V = table.shape[0]
    NW = 32
    neg_flat = x_negative.reshape(B * NUM_NEG)
    nfull = V // SB
    tl = table[nfull * SB :, :]
    j = jnp.arange(tl.shape[0] // 2)[:, None, None]
    p = jnp.arange(2)[None, :, None]
    d = jnp.arange(DIM)[None, None, :]
    vloc = 2 * j + p
    col = p * DIM + (d ^ (vloc & 15))
    vals = tl[vloc, d]
    tail = (
        jnp.zeros((tl.shape[0] // 2, 2 * DIM), jnp.float32)
        .at[j, col]
        .set(vals)
    )
    table2 = _build_sc_repack(V, NW)(table.T, tail)
    table3 = table2.reshape(V, DIM)
    flat = _build_sc_kernel(B, NW)(x_center, x_positive, neg_flat, table3)
    return flat.reshape(B, NLOG)


# speedup vs baseline: 1.8509x; 1.8213x over previous
import functools

import jax
import jax.numpy as jnp
from jax import lax
from jax.experimental import pallas as pl
from jax.experimental.pallas import tpu as pltpu
from jax.experimental.pallas import tpu_sc as plsc

DIM = 64
NUM_NEG = 5
NLOG = 1 + NUM_NEG
CHUNK = 128
LANES = 16
VB = 128

_SC_PARAMS = pltpu.CompilerParams(
    use_tc_tiling_on_sc=True, needs_layout_passes=False
)


SB = 384


@functools.lru_cache(maxsize=None)
def _build_sc_repack(V: int, NW: int):
    nfull = V // SB
    per_w = (nfull + NW - 1) // NW
    rows_out = V // 2
    tail_rows = (V - nfull * SB) // 2
    mesh = plsc.VectorSubcoreMesh(core_axis_name="c", subcore_axis_name="s")

    @functools.partial(
        pl.kernel,
        mesh=mesh,
        compiler_params=_SC_PARAMS,
        out_type=jax.ShapeDtypeStruct((rows_out, 2 * DIM), jnp.float32),
        scratch_types=[
            pltpu.VMEM((DIM, SB), jnp.float32),
            pltpu.VMEM((DIM, SB), jnp.float32),
            pltpu.VMEM((SB // 2, 2 * DIM), jnp.float32),
            pltpu.VMEM((SB // 2, 2 * DIM), jnp.float32),
            pltpu.VMEM((tail_rows, 2 * DIM), jnp.float32),
            pltpu.SemaphoreType.DMA,
            pltpu.SemaphoreType.DMA,
            pltpu.SemaphoreType.DMA,
            pltpu.SemaphoreType.DMA,
        ],
    )
    def repack(
        tt_hbm, tail_hbm, out_hbm,
        st_a, st_b, ob_a, ob_b, tbuf, sin_a, sin_b, sout_a, sout_b,
    ):
        wid = lax.axis_index("s") * 2 + lax.axis_index("c")
        lane = lax.iota(jnp.int32, LANES)
        base_c = wid * per_w
        my_n = jnp.maximum(0, jnp.minimum(per_w, nfull - base_c))

        def in_cps(t, buf, sem):
            c0 = (base_c + t) * SB
            return [
                pltpu.make_async_copy(
                    tt_hbm.at[pl.ds(h * (DIM // 2), DIM // 2),
                              pl.ds(c0, SB)],
                    buf.at[pl.ds(h * (DIM // 2), DIM // 2), :],
                    sem,
                )
                for h in range(2)
            ]

        def out_cps(t, buf, sem):
            r0 = (base_c + t) * (SB // 2)
            return [
                pltpu.make_async_copy(
                    buf.at[pl.ds(h * (SB // 4), SB // 4), :],
                    out_hbm.at[pl.ds(r0 + h * (SB // 4), SB // 4), :],
                    sem,
                )
                for h in range(2)
            ]

        def start_all(cps):
            for cp in cps:
                cp.start()

        def wait_all(cps):
            for cp in cps:
                cp.wait()

        rows_m = []
        par64_m = []
        key_m = []
        for m in range(SB // LANES):
            c = m * LANES + lane
            rows_m.append(c >> 1)
            par64_m.append((c & 1) * DIM)
            key_m.append(c & 15)

        def transpose_block(stage, obuf):
            @plsc.parallel_loop(0, DIM, unroll=8)
            def dstep(d):
                for m in range(SB // LANES):
                    v = stage[d, pl.ds(m * LANES, LANES)]
                    col = par64_m[m] + (d ^ key_m[m])
                    plsc.store_scatter(obuf, [rows_m[m], col], v)

        @pl.when(0 < my_n)
        def _():
            start_all(in_cps(0, st_a, sin_a))

        def halfstep(t, stage, obuf, sin, sout, nstage, nsin):
            @pl.when(t < my_n)
            def _():
                wait_all(in_cps(t, stage, sin))

                @pl.when(t + 1 < my_n)
                def _():
                    start_all(in_cps(t + 1, nstage, nsin))

                @pl.when(t >= 2)
                def _():
                    wait_all(out_cps(t - 2, obuf, sout))

                transpose_block(stage, obuf)
                start_all(out_cps(t, obuf, sout))

        def step(u, _):
            halfstep(2 * u, st_a, ob_a, sin_a, sout_a, st_b, sin_b)
            halfstep(2 * u + 1, st_b, ob_b, sin_b, sout_b, st_a, sin_a)
            return 0

        lax.fori_loop(0, (per_w + 1) // 2, step, 0)

        def drain(t):
            @pl.when((t >= 0) & (t % 2 == 0))
            def _():
                wait_all(out_cps(t, ob_a, sout_a))

            @pl.when((t >= 0) & (t % 2 == 1))
            def _():
                wait_all(out_cps(t, ob_b, sout_b))

        drain(my_n - 2)
        drain(my_n - 1)

        @pl.when(wid == NW - 1)
        def _():
            pltpu.sync_copy(tail_hbm, tbuf)
            pltpu.sync_copy(
                tbuf, out_hbm.at[pl.ds(nfull * (SB // 2), tail_rows), :]
            )

    return repack


@functools.lru_cache(maxsize=None)
def _build_sc_kernel(B: int, NW: int):
    b_per_w = B // NW
    n_chunks = b_per_w // CHUNK
    mesh = plsc.VectorSubcoreMesh(core_axis_name="c", subcore_axis_name="s")

    @functools.partial(
        pl.kernel,
        mesh=mesh,
        compiler_params=pltpu.CompilerParams(
            use_tc_tiling_on_sc=False, needs_layout_passes=False
        ),
        out_type=jax.ShapeDtypeStruct((B * NLOG,), jnp.float32),
        scratch_types=[
            pltpu.VMEM((CHUNK,), jnp.int32),
            pltpu.VMEM((CHUNK,), jnp.int32),
            pltpu.VMEM((CHUNK * NUM_NEG,), jnp.int32),
            pltpu.VMEM((CHUNK, DIM), jnp.float32),
            pltpu.VMEM((CHUNK, DIM), jnp.float32),
            pltpu.VMEM((CHUNK * NUM_NEG, DIM), jnp.float32),
            pltpu.VMEM((CHUNK * NLOG,), jnp.float32),
            pltpu.SemaphoreType.DMA,
        ],
    )
    def sc_kernel(
        cen_hbm, pos_hbm, neg_hbm, table_hbm, out_hbm,
        idc, idp, idn, rows_c, rows_p, rows_n, out_v, sem,
    ):
        wid = lax.axis_index("s") * 2 + lax.axis_index("c")
        lane = lax.iota(jnp.int32, LANES)

        for c in range(n_chunks):
            base = wid * b_per_w + c * CHUNK
            pltpu.sync_copy(cen_hbm.at[pl.ds(base, CHUNK)], idc)
            pltpu.sync_copy(pos_hbm.at[pl.ds(base, CHUNK)], idp)
            pltpu.sync_copy(
                neg_hbm.at[pl.ds(base * NUM_NEG, CHUNK * NUM_NEG)], idn
            )

            cps = [
                pltpu.async_copy(table_hbm.at[idc], rows_c, sem),
                pltpu.async_copy(table_hbm.at[idp], rows_p, sem),
            ] + [
                pltpu.async_copy(
                    table_hbm.at[idn.at[pl.ds(g * CHUNK, CHUNK)]],
                    rows_n.at[pl.ds(g * CHUNK, CHUNK)],
                    sem,
                )
                for g in range(NUM_NEG)
            ]
            for cp in cps:
                cp.wait()

            def group(g, _):
                bvec = g * LANES + lane
                s = pl.ds(g * LANES, LANES)
                w_c = idc[s] & 15
                w_p = idp[s] & 15
                nvecs = [bvec * NUM_NEG + j for j in range(NUM_NEG)]
                w_n = [
                    plsc.load_gather(idn, [nvecs[j]]) & 15
                    for j in range(NUM_NEG)
                ]
                acc = [jnp.zeros((LANES,), jnp.float32) for _ in range(NLOG)]
                for d in range(DIM):
                    cen = plsc.load_gather(rows_c, [bvec, d ^ w_c])
                    acc[0] = acc[0] + cen * plsc.load_gather(
                        rows_p, [bvec, d ^ w_p]
                    )
                    for j in range(NUM_NEG):
                        acc[1 + j] = acc[1 + j] + cen * plsc.load_gather(
                            rows_n, [nvecs[j], d ^ w_n[j]]
                        )
                for j in range(NLOG):
                    prob = 1.0 / (1.0 + jnp.exp(-acc[j]))
                    plsc.store_scatter(out_v, [bvec * NLOG + j], prob)
                return 0

            lax.fori_loop(0, CHUNK // LANES, group, 0)

            pltpu.sync_copy(out_v, out_hbm.at[pl.ds(base * NLOG, CHUNK * NLOG)])

    return sc_kernel


def kernel(x_center, x_positive, x_negative, table):
    B = x_center.shape[0]
    V = table.shape[0]
    NW = 32
    neg_flat = x_negative.reshape(B * NUM_NEG)
    nfull = V // SB
    tl = table[nfull * SB :, :]
    j = jnp.arange(tl.shape[0] // 2)[:, None, None]
    p = jnp.arange(2)[None, :, None]
    d = jnp.arange(DIM)[None, None, :]
    vloc = 2 * j + p
    col = p * DIM + (d ^ (vloc & 15))
    vals = tl[vloc, d]
    tail = (
        jnp.zeros((tl.shape[0] // 2, 2 * DIM), jnp.float32)
        .at[j, col]
        .set(vals)
    )
    table2 = _build_sc_repack(V, NW)(table.T, tail)
    table3 = table2.reshape(V, DIM)
    flat = _build_sc_kernel(B, NW)(x_center, x_positive, neg_flat, table3)
    return flat.reshape(B, NLOG)
